# baseline (speedup 1.0000x reference)
import jax
import jax.numpy as jnp
from jax.experimental import pallas as pl
from jax.experimental.pallas import tpu as pltpu

H = 139
V = 53
L = 6

_DN_MINOR = (((1,), (1,)), ((), ()))


def _gru_body(inp_ref, hidden_ref, emb_ref, wih_ref, whh_ref, bih_ref,
              bhh_ref, wdec_ref, bdec_ref, out_ref, hout_ref):
    idx = inp_ref[0]
    x = emb_ref[pl.ds(idx, 1), :]
    gh = []
    for l in range(L):
        g = jax.lax.dot_general(hidden_ref[l], whh_ref[l], _DN_MINOR,
                                preferred_element_type=jnp.float32)
        gh.append(g + bhh_ref[l])
    for l in range(L):
        h = hidden_ref[l]
        gi = jax.lax.dot_general(x, wih_ref[l], _DN_MINOR,
                                 preferred_element_type=jnp.float32)
        gi = gi + bih_ref[l]
        ghl = gh[l]
        r = jax.nn.sigmoid(gi[:, :H] + ghl[:, :H])
        z = jax.nn.sigmoid(gi[:, H:2 * H] + ghl[:, H:2 * H])
        n = jnp.tanh(gi[:, 2 * H:] + r * ghl[:, 2 * H:])
        x = (1.0 - z) * n + z * h
        hout_ref[l] = x
    out = jax.lax.dot_general(x, wdec_ref[...], _DN_MINOR,
                              preferred_element_type=jnp.float32)
    out_ref[...] = out + bdec_ref[...]


def kernel(input, hidden, emb, W_ih, W_hh, b_ih, b_hh, W_dec, b_dec):
    bih = b_ih.reshape(L, 1, 3 * H)
    bhh = b_hh.reshape(L, 1, 3 * H)
    bdec = b_dec.reshape(1, V)
    idx = input.astype(jnp.int32)

    out, hout = pl.pallas_call(
        _gru_body,
        out_shape=[
            jax.ShapeDtypeStruct((1, V), jnp.float32),
            jax.ShapeDtypeStruct((L, 1, H), jnp.float32),
        ],
        in_specs=[
            pl.BlockSpec(memory_space=pltpu.SMEM),
            pl.BlockSpec(memory_space=pltpu.VMEM),
            pl.BlockSpec(memory_space=pltpu.VMEM),
            pl.BlockSpec(memory_space=pltpu.VMEM),
            pl.BlockSpec(memory_space=pltpu.VMEM),
            pl.BlockSpec(memory_space=pltpu.VMEM),
            pl.BlockSpec(memory_space=pltpu.VMEM),
            pl.BlockSpec(memory_space=pltpu.VMEM),
            pl.BlockSpec(memory_space=pltpu.VMEM),
        ],
        out_specs=[
            pl.BlockSpec(memory_space=pltpu.VMEM),
            pl.BlockSpec(memory_space=pltpu.VMEM),
        ],
    )(idx, hidden, emb, W_ih, W_hh, bih, bhh, W_dec, bdec)
    return out, hout


# speedup vs baseline: 1.4442x; 1.4442x over previous
import jax
import jax.numpy as jnp
from jax.experimental import pallas as pl
from jax.experimental.pallas import tpu as pltpu

H = 139
V = 53
L = 6


def _gru_body(inp_ref, hidden_ref, emb_ref, wih_ref, whh_ref, bih_ref,
              bhh_ref, wdec_ref, bdec_ref, out_ref, hout_ref):
    idx = inp_ref[0]
    x = emb_ref[pl.ds(idx, 1), :]
    gh = []
    for l in range(L):
        g = jnp.dot(hidden_ref[l], whh_ref[l],
                    preferred_element_type=jnp.float32)
        gh.append(g + bhh_ref[l])
    for l in range(L):
        h = hidden_ref[l]
        gi = jnp.dot(x, wih_ref[l], preferred_element_type=jnp.float32)
        gi = gi + bih_ref[l]
        ghl = gh[l]
        r = jax.nn.sigmoid(gi[:, :H] + ghl[:, :H])
        z = jax.nn.sigmoid(gi[:, H:2 * H] + ghl[:, H:2 * H])
        n = jnp.tanh(gi[:, 2 * H:] + r * ghl[:, 2 * H:])
        x = (1.0 - z) * n + z * h
        hout_ref[l] = x
    out = jnp.dot(x, wdec_ref[...], preferred_element_type=jnp.float32)
    out_ref[...] = out + bdec_ref[...]


def kernel(input, hidden, emb, W_ih, W_hh, b_ih, b_hh, W_dec, b_dec):
    wih_t = W_ih.transpose(0, 2, 1)
    whh_t = W_hh.transpose(0, 2, 1)
    bih = b_ih.reshape(L, 1, 3 * H)
    bhh = b_hh.reshape(L, 1, 3 * H)
    wdec_t = W_dec.T
    bdec = b_dec.reshape(1, V)
    idx = input.astype(jnp.int32)

    out, hout = pl.pallas_call(
        _gru_body,
        out_shape=[
            jax.ShapeDtypeStruct((1, V), jnp.float32),
            jax.ShapeDtypeStruct((L, 1, H), jnp.float32),
        ],
        in_specs=[
            pl.BlockSpec(memory_space=pltpu.SMEM),
            pl.BlockSpec(memory_space=pltpu.VMEM),
            pl.BlockSpec(memory_space=pltpu.VMEM),
            pl.BlockSpec(memory_space=pltpu.VMEM),
            pl.BlockSpec(memory_space=pltpu.VMEM),
            pl.BlockSpec(memory_space=pltpu.VMEM),
            pl.BlockSpec(memory_space=pltpu.VMEM),
            pl.BlockSpec(memory_space=pltpu.VMEM),
            pl.BlockSpec(memory_space=pltpu.VMEM),
        ],
        out_specs=[
            pl.BlockSpec(memory_space=pltpu.VMEM),
            pl.BlockSpec(memory_space=pltpu.VMEM),
        ],
    )(idx, hidden, emb, wih_t, whh_t, bih, bhh, wdec_t, bdec)
    return out, hout
